# baseline (speedup 1.0000x reference)
import numpy as np
import jax
import jax.numpy as jnp
from jax import lax
from jax.experimental import pallas as pl
from jax.experimental.pallas import tpu as pltpu
from jax.experimental.pallas import tpu_sc as plsc

_N = 1024
_H = 32
_SCALES = (0.5, 1.0, 2.0)
_K_VALUES = (5, 10, 25)
_KMAX = 25
_BI = 16


def _safe_sqrt(s):
    pos = s > 0
    return jnp.where(pos, jnp.sqrt(jnp.where(pos, s, 1.0)), 0.0)


def _row_normalize(x):
    n = jnp.sqrt(jnp.sum(x * x, axis=1, keepdims=True))
    return x / jnp.maximum(n, 1e-12)


def _feat_body(pr_ref, pc_ref, wm_ref, bm_ref, wl_ref, bl_ref, out_ref):
    n = _N
    xc = pr_ref[:, 0:1]
    yc = pr_ref[:, 1:2]
    zc = pr_ref[:, 2:3]
    xr = pc_ref[0:1, :]
    yr = pc_ref[1:2, :]
    zr = pc_ref[2:3, :]
    dx = xc - xr
    dy = yc - yr
    dz = zc - zr
    sq = dx * dx + dy * dy + dz * dz
    dist = _safe_sqrt(sq)

    dens_cols = []
    grad_cols = []
    for r in _SCALES:
        vol = 4.0 / 3.0 * np.pi * r ** 3
        w = (dist < r).astype(jnp.float32)
        dcol = jnp.sum(w, axis=1, keepdims=True) / vol
        drow = jnp.sum(w, axis=0, keepdims=True) / vol
        dd = dcol - drow
        grad_cols.append(_safe_sqrt(jnp.sum(dd * dd, axis=1, keepdims=True)))
        dens_cols.append(dcol)
    mdve = jnp.concatenate(dens_cols + grad_cols, axis=1)

    lane = lax.broadcasted_iota(jnp.int32, (n, n), 1)
    rowi = lax.broadcasted_iota(jnp.int32, (n, n), 0)
    big = jnp.float32(3.0e38)
    d = jnp.where(lane == rowi, big, dist)
    vals = []
    for t in range(_KMAX):
        m = jnp.min(d, axis=1, keepdims=True)
        vals.append(m)
        if t < _KMAX - 1:
            first = jnp.min(jnp.where(d == m, lane, n), axis=1, keepdims=True)
            d = jnp.where(lane == first, big, d)

    ent_cols = []
    for k in _K_VALUES:
        kn = jnp.concatenate(vals[:k], axis=1)
        s = jnp.sum(kn, axis=1, keepdims=True)
        p = kn / s
        ent_cols.append(-jnp.sum(p * jnp.log(p + 1e-10), axis=1, keepdims=True))
    lgee = jnp.concatenate(ent_cols, axis=1)

    mdve_n = _row_normalize(mdve)
    lgee_n = _row_normalize(lgee)

    memb = bm_ref[0:1, :]
    for c in range(6):
        memb = memb + mdve_n[:, c:c + 1] * wm_ref[c:c + 1, :]
    lemb = bl_ref[0:1, :]
    for c in range(3):
        lemb = lemb + lgee_n[:, c:c + 1] * wl_ref[c:c + 1, :]

    out_ref[...] = _row_normalize(memb) + _row_normalize(lemb)


def _bcast_body(comb_ref, out_hbm, scratch, sem):
    scratch[...] = jnp.broadcast_to(comb_ref[...][None], scratch.shape)
    cps = []
    for c in range(_N // _BI):
        cp = pltpu.make_async_copy(
            scratch, out_hbm.at[0, pl.ds(c * _BI, _BI), :, :], sem)
        cp.start()
        cps.append(cp)
    for cp in cps:
        cp.wait()


def _bcast_body(comb_any, out_hbm, sem):
    cp = pltpu.make_async_copy(comb_any, out_hbm.at[0, 0], sem)
    cp.start()
    cp.wait()
    k = 1
    while k < 64:
        cp = pltpu.make_async_copy(
            out_hbm.at[0, pl.ds(0, k)], out_hbm.at[0, pl.ds(k, k)], sem)
        cp.start()
        cp.wait()
        k *= 2
    cps = []
    for t in range(1, _N // 64):
        cp = pltpu.make_async_copy(
            out_hbm.at[0, pl.ds(0, 64)], out_hbm.at[0, pl.ds(64 * t, 64)],
            sem)
        cp.start()
        cps.append(cp)
    for cp in cps:
        cp.wait()


def kernel(points, W_rtdie, b_rtdie, W_mdve, b_mdve, W_lgee, b_lgee):
    n, h = _N, _H
    p = points[0].astype(jnp.float32)
    pr = jnp.zeros((n, 8), jnp.float32).at[:, 0:3].set(p)
    pc = jnp.zeros((8, n), jnp.float32).at[0:3, :].set(p.T)
    wm = jnp.zeros((8, h), jnp.float32).at[0:6, :].set(W_mdve.T)
    wl = jnp.zeros((8, h), jnp.float32).at[0:3, :].set(W_lgee.T)
    bm = b_mdve.reshape(1, h)
    bl = b_lgee.reshape(1, h)

    comb = pl.pallas_call(
        _feat_body,
        out_shape=jax.ShapeDtypeStruct((n, h), jnp.float32),
        compiler_params=pltpu.CompilerParams(
            vmem_limit_bytes=100 * 1024 * 1024),
    )(pr, pc, wm, bm, wl, bl)

    big = pl.pallas_call(
        _bcast_body,
        in_specs=[pl.BlockSpec(memory_space=pl.MemorySpace.ANY)],
        out_specs=pl.BlockSpec(memory_space=pl.MemorySpace.ANY),
        out_shape=jax.ShapeDtypeStruct((1, n, n, h), jnp.float32),
        scratch_shapes=[pltpu.SemaphoreType.DMA],
    )(comb)
    return big


# speedup vs baseline: 180.6528x; 180.6528x over previous
import numpy as np
import jax
import jax.numpy as jnp
from jax import lax
from jax.experimental import pallas as pl
from jax.experimental.pallas import tpu as pltpu
from jax.experimental.pallas import tpu_sc as plsc

_N = 1024
_H = 32
_SCALES = (0.5, 1.0, 2.0)
_K_VALUES = (5, 10, 25)
_KMAX = 25
_BI = 16


def _safe_sqrt(s):
    pos = s > 0
    return jnp.where(pos, jnp.sqrt(jnp.where(pos, s, 1.0)), 0.0)


def _row_normalize(x):
    n = jnp.sqrt(jnp.sum(x * x, axis=1, keepdims=True))
    return x / jnp.maximum(n, 1e-12)


def _feat_body(pr_ref, pc_ref, wm_ref, bm_ref, wl_ref, bl_ref, out_ref):
    n = _N
    xc = pr_ref[:, 0:1]
    yc = pr_ref[:, 1:2]
    zc = pr_ref[:, 2:3]
    xr = pc_ref[0:1, :]
    yr = pc_ref[1:2, :]
    zr = pc_ref[2:3, :]
    dx = xc - xr
    dy = yc - yr
    dz = zc - zr
    sq = dx * dx + dy * dy + dz * dz
    dist = _safe_sqrt(sq)

    dens_cols = []
    grad_cols = []
    for r in _SCALES:
        vol = 4.0 / 3.0 * np.pi * r ** 3
        w = (dist < r).astype(jnp.float32)
        dcol = jnp.sum(w, axis=1, keepdims=True) / vol
        drow = jnp.sum(w, axis=0, keepdims=True) / vol
        dd = dcol - drow
        grad_cols.append(_safe_sqrt(jnp.sum(dd * dd, axis=1, keepdims=True)))
        dens_cols.append(dcol)
    mdve = jnp.concatenate(dens_cols + grad_cols, axis=1)

    lane = lax.broadcasted_iota(jnp.int32, (n, n), 1)
    rowi = lax.broadcasted_iota(jnp.int32, (n, n), 0)
    big = jnp.float32(3.0e38)
    d = jnp.where(lane == rowi, big, dist)
    vals = []
    for t in range(_KMAX):
        m = jnp.min(d, axis=1, keepdims=True)
        vals.append(m)
        if t < _KMAX - 1:
            first = jnp.min(jnp.where(d == m, lane, n), axis=1, keepdims=True)
            d = jnp.where(lane == first, big, d)

    ent_cols = []
    for k in _K_VALUES:
        kn = jnp.concatenate(vals[:k], axis=1)
        s = jnp.sum(kn, axis=1, keepdims=True)
        p = kn / s
        ent_cols.append(-jnp.sum(p * jnp.log(p + 1e-10), axis=1, keepdims=True))
    lgee = jnp.concatenate(ent_cols, axis=1)

    mdve_n = _row_normalize(mdve)
    lgee_n = _row_normalize(lgee)

    memb = bm_ref[0:1, :]
    for c in range(6):
        memb = memb + mdve_n[:, c:c + 1] * wm_ref[c:c + 1, :]
    lemb = bl_ref[0:1, :]
    for c in range(3):
        lemb = lemb + lgee_n[:, c:c + 1] * wl_ref[c:c + 1, :]

    out_ref[...] = _row_normalize(memb) + _row_normalize(lemb)


def _bcast_body(comb_ref, out_hbm, scratch, sem):
    scratch[...] = jnp.broadcast_to(comb_ref[...][None], scratch.shape)
    cps = []
    for c in range(_N // _BI):
        cp = pltpu.make_async_copy(
            scratch, out_hbm.at[0, pl.ds(c * _BI, _BI), :, :], sem)
        cp.start()
        cps.append(cp)
    for cp in cps:
        cp.wait()


def _bcast_body(comb_any, out_hbm, sem):
    cp = pltpu.make_async_copy(comb_any, out_hbm.at[0, 0], sem)
    cp.start()
    cp.wait()
    k = 1
    while k < 64:
        cp = pltpu.make_async_copy(
            out_hbm.at[0, pl.ds(0, k)], out_hbm.at[0, pl.ds(k, k)], sem)
        cp.start()
        cp.wait()
        k *= 2
    cps = []
    for t in range(1, _N // 64):
        cp = pltpu.make_async_copy(
            out_hbm.at[0, pl.ds(0, 64)], out_hbm.at[0, pl.ds(64 * t, 64)],
            sem)
        cp.start()
        cps.append(cp)
    for cp in cps:
        cp.wait()


def kernel(points, W_rtdie, b_rtdie, W_mdve, b_mdve, W_lgee, b_lgee):
    n, h = _N, _H
    p = points[0].astype(jnp.float32)
    pr = jnp.zeros((n, 8), jnp.float32).at[:, 0:3].set(p)
    pc = jnp.zeros((8, n), jnp.float32).at[0:3, :].set(p.T)
    wm = jnp.zeros((8, h), jnp.float32).at[0:6, :].set(W_mdve.T)
    wl = jnp.zeros((8, h), jnp.float32).at[0:3, :].set(W_lgee.T)
    bm = b_mdve.reshape(1, h)
    bl = b_lgee.reshape(1, h)

    comb = pl.pallas_call(
        _feat_body,
        out_shape=jax.ShapeDtypeStruct((n, h), jnp.float32),
        compiler_params=pltpu.CompilerParams(
            vmem_limit_bytes=100 * 1024 * 1024),
    )(pr, pc, wm, bm, wl, bl)

    return jnp.broadcast_to(comb[None, None], (1, n, n, h))
